# baseline (speedup 1.0000x reference)
import functools

import jax
import jax.numpy as jnp
from jax import lax
from jax.experimental import pallas as pl
from jax.experimental.pallas import tpu as pltpu
from jax.experimental.pallas import tpu_sc as plsc

N_NODES = 10000
N_EDGES = 160000
NFEAT = 256
NHID = 1024
NCLASS = 64
NUM_GRAPHS = 64

NC, NS, LANES = 2, 16, 16

EPAD = 163840
EW = 32
ER = EPAD // EW
RPT = ER // NS
RPW = ER // (NC * NS)
BR = 32
DEPTH = 8
NCHUNK = NHID // 128
CPS = NCHUNK // NC
NP = 10240
NROWS_T = NP // NS
ZR = 128
DEGP = 10240
DPT = DEGP // NS

MB = 1000

_sc_mesh = plsc.VectorSubcoreMesh(
    core_axis_name="c", subcore_axis_name="s", num_cores=NC, num_subcores=NS
)


def _sc_deg_body(dst_hbm, ew_hbm, out_hbm, accd, dstb, ewb, zbuf):
    c = lax.axis_index("c")
    s = lax.axis_index("s")
    wid = s * NC + c

    def _z(i, _):
        zbuf[pl.ds(i * LANES, LANES)] = jnp.zeros((LANES,), jnp.float32)
        return 0

    lax.fori_loop(0, DPT // LANES, _z, 0)
    pltpu.sync_copy(zbuf, accd.at[pl.ds(s * DPT, DPT)])
    plsc.subcore_barrier()

    pltpu.sync_copy(dst_hbm.at[pl.ds(wid * RPW, RPW)], dstb)
    pltpu.sync_copy(ew_hbm.at[pl.ds(wid * RPW, RPW)], ewb)

    def _row(r, _):
        pltpu.sync_copy(ewb.at[r], accd.at[dstb.at[r]], add=True)
        return 0

    lax.fori_loop(0, RPW, _row, 0)
    plsc.subcore_barrier()
    pltpu.sync_copy(accd.at[pl.ds(s * DPT, DPT)], out_hbm.at[c, pl.ds(s * DPT, DPT)])


@functools.partial(
    pl.kernel,
    out_type=jax.ShapeDtypeStruct((NC, DEGP), jnp.float32),
    mesh=_sc_mesh,
    scratch_types=[
        pltpu.VMEM_SHARED((DEGP,), jnp.float32),
        pltpu.VMEM((RPW, EW), jnp.int32),
        pltpu.VMEM((RPW, EW), jnp.float32),
        pltpu.VMEM((DPT,), jnp.float32),
    ],
)
def _sc_deg(dst_hbm, ew_hbm, out_hbm, accd, dstb, ewb, zbuf):
    _sc_deg_body(dst_hbm, ew_hbm, out_hbm, accd, dstb, ewb, zbuf)


_BCAST_DNUMS = lax.GatherDimensionNumbers(
    offset_dims=(), collapsed_slice_dims=(0,), start_index_map=(0,)
)


def _lane_bcast(v16, lane):
    idx = jnp.broadcast_to(lane, (LANES, 1)).astype(jnp.int32)
    return lax.gather(v16, idx, _BCAST_DNUMS, (1,),
                      mode=lax.GatherScatterMode.PROMISE_IN_BOUNDS)


def _sc_agg_body(zs_hbm, src_hbm, dst_hbm, ew_hbm, out_hbm,
                 acc, srcb, dstb, ewb, *bufs_and_sems):
    bufs = bufs_and_sems[:DEPTH]
    gsems = bufs_and_sems[DEPTH:2 * DEPTH]
    ssems = bufs_and_sems[2 * DEPTH:]
    c = lax.axis_index("c")
    s = lax.axis_index("s")

    def _scale(buf, ewblk, r):
        def _scale16(eb, _):
            w16 = ewblk[r, pl.ds(eb * LANES, LANES)]
            for l in range(LANES):
                wf = _lane_bcast(w16, l)
                e = eb * LANES + l
                for j in range(8):
                    sl = pl.ds(j * LANES, LANES)
                    buf[e, sl] = buf[e, sl] * wf
            return 0

        lax.fori_loop(0, EW // LANES, _scale16, 0)

    for ci in range(CPS):
        chunk = c * CPS + ci
        base = chunk * NP

        def _z(i, _):
            bufs[0][i // 8, pl.ds((i % 8) * LANES, LANES)] = jnp.zeros(
                (LANES,), jnp.float32)
            return 0

        lax.fori_loop(0, EW * 8, _z, 0)
        for p in range(NROWS_T // EW):
            pltpu.sync_copy(bufs[0], acc.at[pl.ds(s * NROWS_T + p * EW, EW)])
        plsc.subcore_barrier()

        def _blk(bi, _):
            gr0 = s * RPT + bi * BR
            pltpu.sync_copy(src_hbm.at[chunk, pl.ds(gr0, BR)], srcb)
            pltpu.sync_copy(dst_hbm.at[pl.ds(gr0, BR)], dstb)
            pltpu.sync_copy(ew_hbm.at[pl.ds(gr0, BR)], ewb)

            for b in range(DEPTH - 1):
                pltpu.async_copy(zs_hbm.at[srcb.at[b]], bufs[b], gsems[b])

            def _grp(q, _):
                for b in range(DEPTH):
                    r = DEPTH * q + b
                    nb = (b + DEPTH - 1) % DEPTH
                    pltpu.make_async_copy(
                        zs_hbm.at[srcb.at[r]], bufs[b], gsems[b]).wait()
                    _scale(bufs[b], ewb, r)
                    pltpu.async_copy(
                        bufs[b], acc.at[dstb.at[r]], ssems[b], add=True)

                    @pl.when(r + DEPTH - 1 < BR)
                    def _():
                        @pl.when(r > 0)
                        def _():
                            pltpu.make_async_copy(
                                bufs[nb], acc.at[dstb.at[r - 1]],
                                ssems[nb]).wait()

                        pltpu.async_copy(
                            zs_hbm.at[srcb.at[r + DEPTH - 1]], bufs[nb],
                            gsems[nb])

                return 0

            lax.fori_loop(0, BR // DEPTH, _grp, 0)
            for r in range(BR - DEPTH, BR):
                pltpu.make_async_copy(
                    bufs[r % DEPTH], acc.at[dstb.at[r]],
                    ssems[r % DEPTH]).wait()
            return 0

        lax.fori_loop(0, RPT // BR, _blk, 0)
        plsc.subcore_barrier()

        for p in range(5):
            row = s * NROWS_T + p * ZR
            pltpu.sync_copy(acc.at[pl.ds(row, ZR)],
                            out_hbm.at[pl.ds(base + row, ZR)])
        plsc.subcore_barrier()


@functools.partial(
    pl.kernel,
    out_type=jax.ShapeDtypeStruct((NCHUNK * NP, 128), jnp.float32),
    mesh=_sc_mesh,
    scratch_types=(
        [
            pltpu.VMEM_SHARED((NP, 128), jnp.float32),
            pltpu.VMEM((BR, EW), jnp.int32),
            pltpu.VMEM((BR, EW), jnp.int32),
            pltpu.VMEM((BR, EW), jnp.float32),
        ]
        + [pltpu.VMEM((EW, 128), jnp.float32)] * DEPTH
        + [pltpu.SemaphoreType.DMA] * (2 * DEPTH)
    ),
)
def _sc_agg(zs_hbm, src_hbm, dst_hbm, ew_hbm, out_hbm,
            acc, srcb, dstb, ewb, *bufs_and_sems):
    _sc_agg_body(zs_hbm, src_hbm, dst_hbm, ew_hbm, out_hbm,
                 acc, srcb, dstb, ewb, *bufs_and_sems)


def _mm_scale_kernel(h_ref, w_ref, dinv_ref, out_ref):
    kk = pl.program_id(2)
    nk = pl.num_programs(2)

    @pl.when(kk == 0)
    def _():
        out_ref[...] = jnp.zeros_like(out_ref)

    out_ref[...] += jnp.dot(
        h_ref[...], w_ref[...], preferred_element_type=jnp.float32
    )[None]

    @pl.when(kk == nk - 1)
    def _():
        out_ref[...] *= dinv_ref[...][None]


def _mm_scale(h, W, dinv2d, kb):
    K = h.shape[1]
    grid = (N_NODES // MB, NHID // 128, K // kb)
    return pl.pallas_call(
        _mm_scale_kernel,
        grid=grid,
        in_specs=[
            pl.BlockSpec((MB, kb), lambda m, n, kk: (m, kk)),
            pl.BlockSpec((kb, 128), lambda m, n, kk: (kk, n)),
            pl.BlockSpec((MB, 1), lambda m, n, kk: (m, 0)),
        ],
        out_specs=pl.BlockSpec((1, MB, 128), lambda m, n, kk: (n, m, 0)),
        out_shape=jax.ShapeDtypeStruct((NCHUNK, N_NODES, 128), jnp.float32),
    )(h, W, dinv2d)


def _combine_kernel(agg_ref, zs_ref, dinv_ref, b_ref, out_ref):
    out_ref[...] = jax.nn.relu(
        dinv_ref[...] * (agg_ref[0] + zs_ref[0]) + b_ref[0]
    )


def _combine(agg, zs, dinv2d, b):
    return pl.pallas_call(
        _combine_kernel,
        grid=(N_NODES // MB, NHID // 128),
        in_specs=[
            pl.BlockSpec((1, MB, 128), lambda m, n: (n, m, 0)),
            pl.BlockSpec((1, MB, 128), lambda m, n: (n, m, 0)),
            pl.BlockSpec((MB, 1), lambda m, n: (m, 0)),
            pl.BlockSpec((1, 1, 128), lambda m, n: (n, 0, 0)),
        ],
        out_specs=pl.BlockSpec((MB, 128), lambda m, n: (m, n)),
        out_shape=jax.ShapeDtypeStruct((N_NODES, NHID), jnp.float32),
    )(agg, zs, dinv2d, b.reshape(NCHUNK, 1, 128))


def _pool_final_kernel(h_ref, batch_ref, wf_ref, bf_ref, out_ref, acc_ref, cnt_ref):
    m = pl.program_id(0)
    nm = pl.num_programs(0)

    @pl.when(m == 0)
    def _():
        acc_ref[...] = jnp.zeros_like(acc_ref)
        cnt_ref[...] = jnp.zeros_like(cnt_ref)

    h = h_ref[...]
    b = batch_ref[...]
    gids = jax.lax.broadcasted_iota(jnp.int32, (1, NUM_GRAPHS), 1)
    onehot = (b == gids).astype(jnp.float32)
    acc_ref[...] += jax.lax.dot_general(
        onehot, h, (((0,), (0,)), ((), ())), preferred_element_type=jnp.float32
    )
    cnt_ref[...] += jnp.sum(onehot, axis=0, keepdims=True)

    @pl.when(m == nm - 1)
    def _():
        cnt = jnp.maximum(cnt_ref[...], 1.0)
        g = acc_ref[...] / cnt.reshape(NUM_GRAPHS, 1)
        out_ref[...] = (
            jnp.dot(g, wf_ref[...], preferred_element_type=jnp.float32)
            + bf_ref[...]
        )


def _pool_final(h, batch, Wf, bf):
    return pl.pallas_call(
        _pool_final_kernel,
        grid=(N_NODES // MB,),
        in_specs=[
            pl.BlockSpec((MB, NHID), lambda m: (m, 0)),
            pl.BlockSpec((MB, 1), lambda m: (m, 0)),
            pl.BlockSpec((NHID, NCLASS), lambda m: (0, 0)),
            pl.BlockSpec((1, NCLASS), lambda m: (0, 0)),
        ],
        out_specs=pl.BlockSpec((NUM_GRAPHS, NCLASS), lambda m: (0, 0)),
        out_shape=jax.ShapeDtypeStruct((NUM_GRAPHS, NCLASS), jnp.float32),
        scratch_shapes=[
            pltpu.VMEM((NUM_GRAPHS, NHID), jnp.float32),
            pltpu.VMEM((1, NUM_GRAPHS), jnp.float32),
        ],
    )(h, batch.reshape(N_NODES, 1), Wf, bf.reshape(1, NCLASS))


def kernel(x, edge_index, edge_weight, batch, W1, b1, W2, b2, W3, b3, Wf, bf):
    src, dst = edge_index[0], edge_index[1]
    ew = edge_weight.astype(jnp.float32)

    npad = EPAD - N_EDGES
    fill = (jnp.arange(npad, dtype=jnp.int32) * 37) % N_NODES
    src2d = jnp.concatenate([src, fill]).reshape(ER, EW)
    dst2d = jnp.concatenate([dst, fill]).reshape(ER, EW)
    ew2d = jnp.concatenate([ew, jnp.zeros((npad,), jnp.float32)]).reshape(ER, EW)

    src_off = (src2d[None] +
               (jnp.arange(NCHUNK, dtype=jnp.int32) * N_NODES)[:, None, None])

    degp = _sc_deg(dst2d, ew2d)
    deg = 1.0 + degp[0, :N_NODES] + degp[1, :N_NODES]
    dinv2d = lax.rsqrt(deg).reshape(N_NODES, 1)

    h = x.astype(jnp.float32)
    for W, b, kb in ((W1, b1, 256), (W2, b2, 512), (W3, b3, 512)):
        zs = _mm_scale(h, W, dinv2d, kb)
        agg = _sc_agg(zs.reshape(NCHUNK * N_NODES, 128), src_off, dst2d,
                      ew2d)
        h = _combine(agg.reshape(NCHUNK, NP, 128), zs, dinv2d, b)

    return _pool_final(h, batch, Wf, bf)


# speedup vs baseline: 1.0873x; 1.0873x over previous
import functools

import jax
import jax.numpy as jnp
from jax import lax
from jax.experimental import pallas as pl
from jax.experimental.pallas import tpu as pltpu
from jax.experimental.pallas import tpu_sc as plsc

N_NODES = 10000
N_EDGES = 160000
NFEAT = 256
NHID = 1024
NCLASS = 64
NUM_GRAPHS = 64

NC, NS, LANES = 2, 16, 16

EPAD = 163840
EW = 64
ER = EPAD // EW
RPT = ER // NS
RPW = ER // (NC * NS)
BR = 32
DEPTH = 4
NCHUNK = NHID // 128
CPS = NCHUNK // NC
NP = 10240
NROWS_T = NP // NS
ZR = 128
DEGP = 10240
DPT = DEGP // NS

MB = 1000

_sc_mesh = plsc.VectorSubcoreMesh(
    core_axis_name="c", subcore_axis_name="s", num_cores=NC, num_subcores=NS
)


def _sc_deg_body(dst_hbm, ew_hbm, out_hbm, accd, dstb, ewb, zbuf):
    c = lax.axis_index("c")
    s = lax.axis_index("s")
    wid = s * NC + c

    def _z(i, _):
        zbuf[pl.ds(i * LANES, LANES)] = jnp.zeros((LANES,), jnp.float32)
        return 0

    lax.fori_loop(0, DPT // LANES, _z, 0)
    pltpu.sync_copy(zbuf, accd.at[pl.ds(s * DPT, DPT)])
    plsc.subcore_barrier()

    pltpu.sync_copy(dst_hbm.at[pl.ds(wid * RPW, RPW)], dstb)
    pltpu.sync_copy(ew_hbm.at[pl.ds(wid * RPW, RPW)], ewb)

    def _row(r, _):
        pltpu.sync_copy(ewb.at[r], accd.at[dstb.at[r]], add=True)
        return 0

    lax.fori_loop(0, RPW, _row, 0)
    plsc.subcore_barrier()
    pltpu.sync_copy(accd.at[pl.ds(s * DPT, DPT)], out_hbm.at[c, pl.ds(s * DPT, DPT)])


@functools.partial(
    pl.kernel,
    out_type=jax.ShapeDtypeStruct((NC, DEGP), jnp.float32),
    mesh=_sc_mesh,
    scratch_types=[
        pltpu.VMEM_SHARED((DEGP,), jnp.float32),
        pltpu.VMEM((RPW, EW), jnp.int32),
        pltpu.VMEM((RPW, EW), jnp.float32),
        pltpu.VMEM((DPT,), jnp.float32),
    ],
)
def _sc_deg(dst_hbm, ew_hbm, out_hbm, accd, dstb, ewb, zbuf):
    _sc_deg_body(dst_hbm, ew_hbm, out_hbm, accd, dstb, ewb, zbuf)


_BCAST_DNUMS = lax.GatherDimensionNumbers(
    offset_dims=(), collapsed_slice_dims=(0,), start_index_map=(0,)
)


def _lane_bcast(v16, lane):
    idx = jnp.broadcast_to(lane, (LANES, 1)).astype(jnp.int32)
    return lax.gather(v16, idx, _BCAST_DNUMS, (1,),
                      mode=lax.GatherScatterMode.PROMISE_IN_BOUNDS)


def _sc_agg_body(zs_hbm, src_hbm, dst_hbm, ew_hbm, out_hbm,
                 acc, srcb, dstb, ewb, *bufs_and_sems):
    bufs = bufs_and_sems[:DEPTH]
    gsems = bufs_and_sems[DEPTH:2 * DEPTH]
    ssems = bufs_and_sems[2 * DEPTH:]
    c = lax.axis_index("c")
    s = lax.axis_index("s")

    def _scale(buf, ewblk, r):
        def _scale16(eb, _):
            w16 = ewblk[r, pl.ds(eb * LANES, LANES)]
            for l in range(LANES):
                wf = _lane_bcast(w16, l)
                e = eb * LANES + l
                for j in range(8):
                    sl = pl.ds(j * LANES, LANES)
                    buf[e, sl] = buf[e, sl] * wf
            return 0

        lax.fori_loop(0, EW // LANES, _scale16, 0)

    for ci in range(CPS):
        chunk = c * CPS + ci
        base = chunk * NP

        def _z(i, _):
            bufs[0][i // 8, pl.ds((i % 8) * LANES, LANES)] = jnp.zeros(
                (LANES,), jnp.float32)
            return 0

        lax.fori_loop(0, EW * 8, _z, 0)
        for p in range(NROWS_T // EW):
            pltpu.sync_copy(bufs[0], acc.at[pl.ds(s * NROWS_T + p * EW, EW)])
        plsc.subcore_barrier()

        def _blk(bi, _):
            gr0 = s * RPT + bi * BR
            pltpu.sync_copy(src_hbm.at[chunk, pl.ds(gr0, BR)], srcb)
            pltpu.sync_copy(dst_hbm.at[pl.ds(gr0, BR)], dstb)
            pltpu.sync_copy(ew_hbm.at[pl.ds(gr0, BR)], ewb)

            for b in range(DEPTH - 1):
                pltpu.async_copy(zs_hbm.at[srcb.at[b]], bufs[b], gsems[b])

            def _grp(q, _):
                for b in range(DEPTH):
                    r = DEPTH * q + b
                    nb = (b + DEPTH - 1) % DEPTH
                    pltpu.make_async_copy(
                        zs_hbm.at[srcb.at[r]], bufs[b], gsems[b]).wait()
                    _scale(bufs[b], ewb, r)
                    pltpu.async_copy(
                        bufs[b], acc.at[dstb.at[r]], ssems[b], add=True)

                    @pl.when(r + DEPTH - 1 < BR)
                    def _():
                        @pl.when(r > 0)
                        def _():
                            pltpu.make_async_copy(
                                bufs[nb], acc.at[dstb.at[r - 1]],
                                ssems[nb]).wait()

                        pltpu.async_copy(
                            zs_hbm.at[srcb.at[r + DEPTH - 1]], bufs[nb],
                            gsems[nb])

                return 0

            lax.fori_loop(0, BR // DEPTH, _grp, 0)
            for r in range(BR - DEPTH, BR):
                pltpu.make_async_copy(
                    bufs[r % DEPTH], acc.at[dstb.at[r]],
                    ssems[r % DEPTH]).wait()
            return 0

        lax.fori_loop(0, RPT // BR, _blk, 0)
        plsc.subcore_barrier()

        for p in range(5):
            row = s * NROWS_T + p * ZR
            pltpu.sync_copy(acc.at[pl.ds(row, ZR)],
                            out_hbm.at[pl.ds(base + row, ZR)])
        plsc.subcore_barrier()


@functools.partial(
    pl.kernel,
    out_type=jax.ShapeDtypeStruct((NCHUNK * NP, 128), jnp.float32),
    mesh=_sc_mesh,
    scratch_types=(
        [
            pltpu.VMEM_SHARED((NP, 128), jnp.float32),
            pltpu.VMEM((BR, EW), jnp.int32),
            pltpu.VMEM((BR, EW), jnp.int32),
            pltpu.VMEM((BR, EW), jnp.float32),
        ]
        + [pltpu.VMEM((EW, 128), jnp.float32)] * DEPTH
        + [pltpu.SemaphoreType.DMA] * (2 * DEPTH)
    ),
)
def _sc_agg(zs_hbm, src_hbm, dst_hbm, ew_hbm, out_hbm,
            acc, srcb, dstb, ewb, *bufs_and_sems):
    _sc_agg_body(zs_hbm, src_hbm, dst_hbm, ew_hbm, out_hbm,
                 acc, srcb, dstb, ewb, *bufs_and_sems)


def _mm_scale_kernel(h_ref, w_ref, dinv_ref, out_ref):
    kk = pl.program_id(2)
    nk = pl.num_programs(2)

    @pl.when(kk == 0)
    def _():
        out_ref[...] = jnp.zeros_like(out_ref)

    out_ref[...] += jnp.dot(
        h_ref[...], w_ref[...], preferred_element_type=jnp.float32
    )[None]

    @pl.when(kk == nk - 1)
    def _():
        out_ref[...] *= dinv_ref[...][None]


def _mm_scale(h, W, dinv2d, kb):
    K = h.shape[1]
    grid = (N_NODES // MB, NHID // 128, K // kb)
    return pl.pallas_call(
        _mm_scale_kernel,
        grid=grid,
        in_specs=[
            pl.BlockSpec((MB, kb), lambda m, n, kk: (m, kk)),
            pl.BlockSpec((kb, 128), lambda m, n, kk: (kk, n)),
            pl.BlockSpec((MB, 1), lambda m, n, kk: (m, 0)),
        ],
        out_specs=pl.BlockSpec((1, MB, 128), lambda m, n, kk: (n, m, 0)),
        out_shape=jax.ShapeDtypeStruct((NCHUNK, N_NODES, 128), jnp.float32),
    )(h, W, dinv2d)


def _combine_kernel(agg_ref, zs_ref, dinv_ref, b_ref, out_ref):
    out_ref[...] = jax.nn.relu(
        dinv_ref[...] * (agg_ref[0] + zs_ref[0]) + b_ref[0]
    )


def _combine(agg, zs, dinv2d, b):
    return pl.pallas_call(
        _combine_kernel,
        grid=(N_NODES // MB, NHID // 128),
        in_specs=[
            pl.BlockSpec((1, MB, 128), lambda m, n: (n, m, 0)),
            pl.BlockSpec((1, MB, 128), lambda m, n: (n, m, 0)),
            pl.BlockSpec((MB, 1), lambda m, n: (m, 0)),
            pl.BlockSpec((1, 1, 128), lambda m, n: (n, 0, 0)),
        ],
        out_specs=pl.BlockSpec((MB, 128), lambda m, n: (m, n)),
        out_shape=jax.ShapeDtypeStruct((N_NODES, NHID), jnp.float32),
    )(agg, zs, dinv2d, b.reshape(NCHUNK, 1, 128))


def _pool_final_kernel(h_ref, batch_ref, wf_ref, bf_ref, out_ref, acc_ref, cnt_ref):
    m = pl.program_id(0)
    nm = pl.num_programs(0)

    @pl.when(m == 0)
    def _():
        acc_ref[...] = jnp.zeros_like(acc_ref)
        cnt_ref[...] = jnp.zeros_like(cnt_ref)

    h = h_ref[...]
    b = batch_ref[...]
    gids = jax.lax.broadcasted_iota(jnp.int32, (1, NUM_GRAPHS), 1)
    onehot = (b == gids).astype(jnp.float32)
    acc_ref[...] += jax.lax.dot_general(
        onehot, h, (((0,), (0,)), ((), ())), preferred_element_type=jnp.float32
    )
    cnt_ref[...] += jnp.sum(onehot, axis=0, keepdims=True)

    @pl.when(m == nm - 1)
    def _():
        cnt = jnp.maximum(cnt_ref[...], 1.0)
        g = acc_ref[...] / cnt.reshape(NUM_GRAPHS, 1)
        out_ref[...] = (
            jnp.dot(g, wf_ref[...], preferred_element_type=jnp.float32)
            + bf_ref[...]
        )


def _pool_final(h, batch, Wf, bf):
    return pl.pallas_call(
        _pool_final_kernel,
        grid=(N_NODES // MB,),
        in_specs=[
            pl.BlockSpec((MB, NHID), lambda m: (m, 0)),
            pl.BlockSpec((MB, 1), lambda m: (m, 0)),
            pl.BlockSpec((NHID, NCLASS), lambda m: (0, 0)),
            pl.BlockSpec((1, NCLASS), lambda m: (0, 0)),
        ],
        out_specs=pl.BlockSpec((NUM_GRAPHS, NCLASS), lambda m: (0, 0)),
        out_shape=jax.ShapeDtypeStruct((NUM_GRAPHS, NCLASS), jnp.float32),
        scratch_shapes=[
            pltpu.VMEM((NUM_GRAPHS, NHID), jnp.float32),
            pltpu.VMEM((1, NUM_GRAPHS), jnp.float32),
        ],
    )(h, batch.reshape(N_NODES, 1), Wf, bf.reshape(1, NCLASS))


def kernel(x, edge_index, edge_weight, batch, W1, b1, W2, b2, W3, b3, Wf, bf):
    src, dst = edge_index[0], edge_index[1]
    ew = edge_weight.astype(jnp.float32)

    npad = EPAD - N_EDGES
    fill = (jnp.arange(npad, dtype=jnp.int32) * 37) % N_NODES
    src2d = jnp.concatenate([src, fill]).reshape(ER, EW)
    dst2d = jnp.concatenate([dst, fill]).reshape(ER, EW)
    ew2d = jnp.concatenate([ew, jnp.zeros((npad,), jnp.float32)]).reshape(ER, EW)

    src_off = (src2d[None] +
               (jnp.arange(NCHUNK, dtype=jnp.int32) * N_NODES)[:, None, None])

    degp = _sc_deg(dst2d, ew2d)
    deg = 1.0 + degp[0, :N_NODES] + degp[1, :N_NODES]
    dinv2d = lax.rsqrt(deg).reshape(N_NODES, 1)

    h = x.astype(jnp.float32)
    for W, b, kb in ((W1, b1, 256), (W2, b2, 512), (W3, b3, 512)):
        zs = _mm_scale(h, W, dinv2d, kb)
        agg = _sc_agg(zs.reshape(NCHUNK * N_NODES, 128), src_off, dst2d,
                      ew2d)
        h = _combine(agg.reshape(NCHUNK, NP, 128), zs, dinv2d, b)

    return _pool_final(h, batch, Wf, bf)


# speedup vs baseline: 1.1998x; 1.1034x over previous
import functools

import jax
import jax.numpy as jnp
from jax import lax
from jax.experimental import pallas as pl
from jax.experimental.pallas import tpu as pltpu
from jax.experimental.pallas import tpu_sc as plsc

N_NODES = 10000
N_EDGES = 160000
NFEAT = 256
NHID = 1024
NCLASS = 64
NUM_GRAPHS = 64

NC, NS, LANES = 2, 16, 16

EPAD = 163840
EW = 64
ER = EPAD // EW
RPT = ER // NS
RPW = ER // (NC * NS)
BR = 32
DEPTH = 4
NCHUNK = NHID // 128
CPS = NCHUNK // NC
NP = 10240
NROWS_T = NP // NS
ZR = 128
DEGP = 10240
DPT = DEGP // NS

MB = 1000

_sc_mesh = plsc.VectorSubcoreMesh(
    core_axis_name="c", subcore_axis_name="s", num_cores=NC, num_subcores=NS
)


def _sc_deg_body(dst_hbm, ew_hbm, out_hbm, accd, dstb, ewb, zbuf):
    c = lax.axis_index("c")
    s = lax.axis_index("s")
    wid = s * NC + c

    def _z(i, _):
        zbuf[pl.ds(i * LANES, LANES)] = jnp.zeros((LANES,), jnp.float32)
        return 0

    lax.fori_loop(0, DPT // LANES, _z, 0)
    pltpu.sync_copy(zbuf, accd.at[pl.ds(s * DPT, DPT)])
    plsc.subcore_barrier()

    pltpu.sync_copy(dst_hbm.at[pl.ds(wid * RPW, RPW)], dstb)
    pltpu.sync_copy(ew_hbm.at[pl.ds(wid * RPW, RPW)], ewb)

    def _row(r, _):
        pltpu.sync_copy(ewb.at[r], accd.at[dstb.at[r]], add=True)
        return 0

    lax.fori_loop(0, RPW, _row, 0)
    plsc.subcore_barrier()
    pltpu.sync_copy(accd.at[pl.ds(s * DPT, DPT)], out_hbm.at[c, pl.ds(s * DPT, DPT)])


@functools.partial(
    pl.kernel,
    out_type=jax.ShapeDtypeStruct((NC, DEGP), jnp.float32),
    mesh=_sc_mesh,
    scratch_types=[
        pltpu.VMEM_SHARED((DEGP,), jnp.float32),
        pltpu.VMEM((RPW, EW), jnp.int32),
        pltpu.VMEM((RPW, EW), jnp.float32),
        pltpu.VMEM((DPT,), jnp.float32),
    ],
)
def _sc_deg(dst_hbm, ew_hbm, out_hbm, accd, dstb, ewb, zbuf):
    _sc_deg_body(dst_hbm, ew_hbm, out_hbm, accd, dstb, ewb, zbuf)


_BCAST_DNUMS = lax.GatherDimensionNumbers(
    offset_dims=(), collapsed_slice_dims=(0,), start_index_map=(0,)
)


def _lane_bcast(v16, lane):
    idx = jnp.broadcast_to(lane, (LANES, 1)).astype(jnp.int32)
    return lax.gather(v16, idx, _BCAST_DNUMS, (1,),
                      mode=lax.GatherScatterMode.PROMISE_IN_BOUNDS)


def _sc_agg_body(zs_hbm, src_hbm, dst_hbm, ew_hbm, out_hbm,
                 acc, srcb, dstb, ewb, *bufs_and_sems):
    bufs = bufs_and_sems[:DEPTH]
    gsems = bufs_and_sems[DEPTH:2 * DEPTH]
    ssems = bufs_and_sems[2 * DEPTH:]
    c = lax.axis_index("c")
    s = lax.axis_index("s")

    def _scale(buf, ewblk, r):
        def _scale16(eb, _):
            w16 = ewblk[r, pl.ds(eb * LANES, LANES)]
            for l in range(LANES):
                wf = _lane_bcast(w16, l)
                e = eb * LANES + l
                for j in range(8):
                    sl = pl.ds(j * LANES, LANES)
                    buf[e, sl] = buf[e, sl] * wf
            return 0

        lax.fori_loop(0, EW // LANES, _scale16, 0)

    for ci in range(CPS):
        chunk = c * CPS + ci
        base = chunk * NP

        def _z(i, _):
            bufs[0][i // 8, pl.ds((i % 8) * LANES, LANES)] = jnp.zeros(
                (LANES,), jnp.float32)
            return 0

        lax.fori_loop(0, EW * 8, _z, 0)
        for p in range(NROWS_T // EW):
            pltpu.sync_copy(bufs[0], acc.at[pl.ds(s * NROWS_T + p * EW, EW)])
        plsc.subcore_barrier()

        def _blk(bi, _):
            gr0 = s * RPT + bi * BR
            pltpu.sync_copy(src_hbm.at[chunk, pl.ds(gr0, BR)], srcb)
            pltpu.sync_copy(dst_hbm.at[pl.ds(gr0, BR)], dstb)
            pltpu.sync_copy(ew_hbm.at[pl.ds(gr0, BR)], ewb)

            for b in range(DEPTH - 1):
                pltpu.async_copy(zs_hbm.at[srcb.at[b]], bufs[b], gsems[b])

            def _grp(q, _):
                for b in range(DEPTH):
                    r = DEPTH * q + b
                    nb = (b + DEPTH - 1) % DEPTH
                    pltpu.make_async_copy(
                        zs_hbm.at[srcb.at[r]], bufs[b], gsems[b]).wait()
                    _scale(bufs[b], ewb, r)
                    pltpu.async_copy(
                        bufs[b], acc.at[dstb.at[r]], ssems[b], add=True)

                    @pl.when(r + DEPTH - 1 < BR)
                    def _():
                        @pl.when(r > 0)
                        def _():
                            pltpu.make_async_copy(
                                bufs[nb], acc.at[dstb.at[r - 1]],
                                ssems[nb]).wait()

                        pltpu.async_copy(
                            zs_hbm.at[srcb.at[r + DEPTH - 1]], bufs[nb],
                            gsems[nb])

                return 0

            lax.fori_loop(0, BR // DEPTH, _grp, 0)
            for r in range(BR - DEPTH, BR):
                pltpu.make_async_copy(
                    bufs[r % DEPTH], acc.at[dstb.at[r]],
                    ssems[r % DEPTH]).wait()
            return 0

        lax.fori_loop(0, RPT // BR, _blk, 0)
        plsc.subcore_barrier()

        for p in range(5):
            row = s * NROWS_T + p * ZR
            pltpu.sync_copy(acc.at[pl.ds(row, ZR)],
                            out_hbm.at[pl.ds(base + row, ZR)])
        plsc.subcore_barrier()


@functools.partial(
    pl.kernel,
    out_type=jax.ShapeDtypeStruct((NCHUNK * NP, 128), jnp.float32),
    mesh=_sc_mesh,
    scratch_types=(
        [
            pltpu.VMEM_SHARED((NP, 128), jnp.float32),
            pltpu.VMEM((BR, EW), jnp.int32),
            pltpu.VMEM((BR, EW), jnp.int32),
            pltpu.VMEM((BR, EW), jnp.float32),
        ]
        + [pltpu.VMEM((EW, 128), jnp.float32)] * DEPTH
        + [pltpu.SemaphoreType.DMA] * (2 * DEPTH)
    ),
)
def _sc_agg(zs_hbm, src_hbm, dst_hbm, ew_hbm, out_hbm,
            acc, srcb, dstb, ewb, *bufs_and_sems):
    _sc_agg_body(zs_hbm, src_hbm, dst_hbm, ew_hbm, out_hbm,
                 acc, srcb, dstb, ewb, *bufs_and_sems)


def _mm_scale_kernel(h_ref, w_ref, dinv_ref, out_ref):
    kk = pl.program_id(2)
    nk = pl.num_programs(2)

    @pl.when(kk == 0)
    def _():
        out_ref[...] = jnp.zeros_like(out_ref)

    out_ref[...] += jnp.dot(
        h_ref[...], w_ref[...], preferred_element_type=jnp.float32
    )[None]

    @pl.when(kk == nk - 1)
    def _():
        out_ref[...] *= dinv_ref[...][None]


def _mm_scale(h, W, dinv2d, kb):
    K = h.shape[1]
    grid = (N_NODES // MB, NHID // 128, K // kb)
    return pl.pallas_call(
        _mm_scale_kernel,
        grid=grid,
        in_specs=[
            pl.BlockSpec((MB, kb), lambda m, n, kk: (m, kk)),
            pl.BlockSpec((kb, 128), lambda m, n, kk: (kk, n)),
            pl.BlockSpec((MB, 1), lambda m, n, kk: (m, 0)),
        ],
        out_specs=pl.BlockSpec((1, MB, 128), lambda m, n, kk: (n, m, 0)),
        out_shape=jax.ShapeDtypeStruct((NCHUNK, N_NODES, 128), jnp.float32),
    )(h, W, dinv2d)


def _combine_kernel(agg_ref, zs_ref, dinv_ref, b_ref, out_ref):
    out_ref[...] = jax.nn.relu(
        dinv_ref[...] * (agg_ref[0] + zs_ref[0]) + b_ref[0]
    )


def _combine(agg, zs, dinv2d, b):
    return pl.pallas_call(
        _combine_kernel,
        grid=(N_NODES // MB, NHID // 128),
        in_specs=[
            pl.BlockSpec((1, MB, 128), lambda m, n: (n, m, 0)),
            pl.BlockSpec((1, MB, 128), lambda m, n: (n, m, 0)),
            pl.BlockSpec((MB, 1), lambda m, n: (m, 0)),
            pl.BlockSpec((1, 1, 128), lambda m, n: (n, 0, 0)),
        ],
        out_specs=pl.BlockSpec((MB, 128), lambda m, n: (m, n)),
        out_shape=jax.ShapeDtypeStruct((N_NODES, NHID), jnp.float32),
    )(agg, zs, dinv2d, b.reshape(NCHUNK, 1, 128))


def _pool_final_kernel(h_ref, batch_ref, wf_ref, bf_ref, out_ref, acc_ref, cnt_ref):
    m = pl.program_id(0)
    nm = pl.num_programs(0)

    @pl.when(m == 0)
    def _():
        acc_ref[...] = jnp.zeros_like(acc_ref)
        cnt_ref[...] = jnp.zeros_like(cnt_ref)

    h = h_ref[...]
    b = batch_ref[...]
    gids = jax.lax.broadcasted_iota(jnp.int32, (1, NUM_GRAPHS), 1)
    onehot = (b == gids).astype(jnp.float32)
    acc_ref[...] += jax.lax.dot_general(
        onehot, h, (((0,), (0,)), ((), ())), preferred_element_type=jnp.float32
    )
    cnt_ref[...] += jnp.sum(onehot, axis=0, keepdims=True)

    @pl.when(m == nm - 1)
    def _():
        cnt = jnp.maximum(cnt_ref[...], 1.0)
        g = acc_ref[...] / cnt.reshape(NUM_GRAPHS, 1)
        out_ref[...] = (
            jnp.dot(g, wf_ref[...], preferred_element_type=jnp.float32)
            + bf_ref[...]
        )


def _pool_final(h, batch, Wf, bf):
    return pl.pallas_call(
        _pool_final_kernel,
        grid=(N_NODES // MB,),
        in_specs=[
            pl.BlockSpec((MB, NHID), lambda m: (m, 0)),
            pl.BlockSpec((MB, 1), lambda m: (m, 0)),
            pl.BlockSpec((NHID, NCLASS), lambda m: (0, 0)),
            pl.BlockSpec((1, NCLASS), lambda m: (0, 0)),
        ],
        out_specs=pl.BlockSpec((NUM_GRAPHS, NCLASS), lambda m: (0, 0)),
        out_shape=jax.ShapeDtypeStruct((NUM_GRAPHS, NCLASS), jnp.float32),
        scratch_shapes=[
            pltpu.VMEM((NUM_GRAPHS, NHID), jnp.float32),
            pltpu.VMEM((1, NUM_GRAPHS), jnp.float32),
        ],
    )(h, batch.reshape(N_NODES, 1), Wf, bf.reshape(1, NCLASS))


def kernel(x, edge_index, edge_weight, batch, W1, b1, W2, b2, W3, b3, Wf, bf):
    src, dst = edge_index[0], edge_index[1]
    ew = edge_weight.astype(jnp.float32)

    npad = EPAD - N_EDGES
    fill = (jnp.arange(npad, dtype=jnp.int32) * 37) % N_NODES
    src2d = jnp.concatenate([src, fill]).reshape(ER, EW)
    dst2d = jnp.concatenate([dst, fill]).reshape(ER, EW)
    ew2d = jnp.concatenate([ew, jnp.zeros((npad,), jnp.float32)]).reshape(ER, EW)

    src_off = (src2d[None] +
               (jnp.arange(NCHUNK, dtype=jnp.int32) * N_NODES)[:, None, None])

    degp = _sc_deg(dst2d, ew2d)
    deg = 1.0 + degp[0, :N_NODES] + degp[1, :N_NODES]
    dinv2d = lax.rsqrt(deg).reshape(N_NODES, 1)

    h = x.astype(jnp.float32)
    for W, b, kb in ((W1, b1, 256), (W2, b2, 1024), (W3, b3, 1024)):
        zs = _mm_scale(h, W, dinv2d, kb)
        agg = _sc_agg(zs.reshape(NCHUNK * N_NODES, 128), src_off, dst2d,
                      ew2d)
        h = _combine(agg.reshape(NCHUNK, NP, 128), zs, dinv2d, b)

    return _pool_final(h, batch, Wf, bf)


# speedup vs baseline: 1.2875x; 1.0731x over previous
import functools

import jax
import jax.numpy as jnp
from jax import lax
from jax.experimental import pallas as pl
from jax.experimental.pallas import tpu as pltpu
from jax.experimental.pallas import tpu_sc as plsc

N_NODES = 10000
N_EDGES = 160000
NFEAT = 256
NHID = 1024
NCLASS = 64
NUM_GRAPHS = 64

NC, NS, LANES = 2, 16, 16

EPAD = 163840
EW = 64
ER = EPAD // EW
RPT = ER // NS
RPW = ER // (NC * NS)
BR = 32
DEPTH = 4
NCHUNK = NHID // 128
CPS = NCHUNK // NC
NP = 10240
NROWS_T = NP // NS
ZR = 128
DEGP = 10240
DPT = DEGP // NS

MB = 2000

_sc_mesh = plsc.VectorSubcoreMesh(
    core_axis_name="c", subcore_axis_name="s", num_cores=NC, num_subcores=NS
)


def _sc_deg_body(dst_hbm, ew_hbm, out_hbm, accd, dstb, ewb, zbuf):
    c = lax.axis_index("c")
    s = lax.axis_index("s")
    wid = s * NC + c

    def _z(i, _):
        zbuf[pl.ds(i * LANES, LANES)] = jnp.zeros((LANES,), jnp.float32)
        return 0

    lax.fori_loop(0, DPT // LANES, _z, 0)
    pltpu.sync_copy(zbuf, accd.at[pl.ds(s * DPT, DPT)])
    plsc.subcore_barrier()

    pltpu.sync_copy(dst_hbm.at[pl.ds(wid * RPW, RPW)], dstb)
    pltpu.sync_copy(ew_hbm.at[pl.ds(wid * RPW, RPW)], ewb)

    def _row(r, _):
        pltpu.sync_copy(ewb.at[r], accd.at[dstb.at[r]], add=True)
        return 0

    lax.fori_loop(0, RPW, _row, 0)
    plsc.subcore_barrier()
    pltpu.sync_copy(accd.at[pl.ds(s * DPT, DPT)], out_hbm.at[c, pl.ds(s * DPT, DPT)])


@functools.partial(
    pl.kernel,
    out_type=jax.ShapeDtypeStruct((NC, DEGP), jnp.float32),
    mesh=_sc_mesh,
    scratch_types=[
        pltpu.VMEM_SHARED((DEGP,), jnp.float32),
        pltpu.VMEM((RPW, EW), jnp.int32),
        pltpu.VMEM((RPW, EW), jnp.float32),
        pltpu.VMEM((DPT,), jnp.float32),
    ],
)
def _sc_deg(dst_hbm, ew_hbm, out_hbm, accd, dstb, ewb, zbuf):
    _sc_deg_body(dst_hbm, ew_hbm, out_hbm, accd, dstb, ewb, zbuf)


_BCAST_DNUMS = lax.GatherDimensionNumbers(
    offset_dims=(), collapsed_slice_dims=(0,), start_index_map=(0,)
)


def _lane_bcast(v16, lane):
    idx = jnp.broadcast_to(lane, (LANES, 1)).astype(jnp.int32)
    return lax.gather(v16, idx, _BCAST_DNUMS, (1,),
                      mode=lax.GatherScatterMode.PROMISE_IN_BOUNDS)


def _sc_agg_body(zs_hbm, src_hbm, dst_hbm, ew_hbm, out_hbm,
                 acc, srcb, dstb, ewb, *bufs_and_sems):
    bufs = bufs_and_sems[:DEPTH]
    gsems = bufs_and_sems[DEPTH:2 * DEPTH]
    ssems = bufs_and_sems[2 * DEPTH:]
    c = lax.axis_index("c")
    s = lax.axis_index("s")

    def _scale(buf, ewblk, r):
        def _scale16(eb, _):
            w16 = ewblk[r, pl.ds(eb * LANES, LANES)]
            for l in range(LANES):
                wf = _lane_bcast(w16, l)
                e = eb * LANES + l
                for j in range(8):
                    sl = pl.ds(j * LANES, LANES)
                    buf[e, sl] = buf[e, sl] * wf
            return 0

        lax.fori_loop(0, EW // LANES, _scale16, 0)

    for ci in range(CPS):
        chunk = c * CPS + ci
        base = chunk * NP

        def _z(i, _):
            bufs[0][i // 8, pl.ds((i % 8) * LANES, LANES)] = jnp.zeros(
                (LANES,), jnp.float32)
            return 0

        lax.fori_loop(0, EW * 8, _z, 0)
        for p in range(NROWS_T // EW):
            pltpu.sync_copy(bufs[0], acc.at[pl.ds(s * NROWS_T + p * EW, EW)])
        plsc.subcore_barrier()

        def _blk(bi, _):
            gr0 = s * RPT + bi * BR
            pltpu.sync_copy(src_hbm.at[chunk, pl.ds(gr0, BR)], srcb)
            pltpu.sync_copy(dst_hbm.at[pl.ds(gr0, BR)], dstb)
            pltpu.sync_copy(ew_hbm.at[pl.ds(gr0, BR)], ewb)

            for b in range(DEPTH - 1):
                pltpu.async_copy(zs_hbm.at[srcb.at[b]], bufs[b], gsems[b])

            def _grp(q, _):
                for b in range(DEPTH):
                    r = DEPTH * q + b
                    nb = (b + DEPTH - 1) % DEPTH
                    pltpu.make_async_copy(
                        zs_hbm.at[srcb.at[r]], bufs[b], gsems[b]).wait()
                    _scale(bufs[b], ewb, r)
                    pltpu.async_copy(
                        bufs[b], acc.at[dstb.at[r]], ssems[b], add=True)

                    @pl.when(r + DEPTH - 1 < BR)
                    def _():
                        @pl.when(r > 0)
                        def _():
                            pltpu.make_async_copy(
                                bufs[nb], acc.at[dstb.at[r - 1]],
                                ssems[nb]).wait()

                        pltpu.async_copy(
                            zs_hbm.at[srcb.at[r + DEPTH - 1]], bufs[nb],
                            gsems[nb])

                return 0

            lax.fori_loop(0, BR // DEPTH, _grp, 0)
            for r in range(BR - DEPTH, BR):
                pltpu.make_async_copy(
                    bufs[r % DEPTH], acc.at[dstb.at[r]],
                    ssems[r % DEPTH]).wait()
            return 0

        lax.fori_loop(0, RPT // BR, _blk, 0)
        plsc.subcore_barrier()

        for p in range(5):
            row = s * NROWS_T + p * ZR
            pltpu.sync_copy(acc.at[pl.ds(row, ZR)],
                            out_hbm.at[pl.ds(base + row, ZR)])
        plsc.subcore_barrier()


@functools.partial(
    pl.kernel,
    out_type=jax.ShapeDtypeStruct((NCHUNK * NP, 128), jnp.float32),
    mesh=_sc_mesh,
    scratch_types=(
        [
            pltpu.VMEM_SHARED((NP, 128), jnp.float32),
            pltpu.VMEM((BR, EW), jnp.int32),
            pltpu.VMEM((BR, EW), jnp.int32),
            pltpu.VMEM((BR, EW), jnp.float32),
        ]
        + [pltpu.VMEM((EW, 128), jnp.float32)] * DEPTH
        + [pltpu.SemaphoreType.DMA] * (2 * DEPTH)
    ),
)
def _sc_agg(zs_hbm, src_hbm, dst_hbm, ew_hbm, out_hbm,
            acc, srcb, dstb, ewb, *bufs_and_sems):
    _sc_agg_body(zs_hbm, src_hbm, dst_hbm, ew_hbm, out_hbm,
                 acc, srcb, dstb, ewb, *bufs_and_sems)


def _mm_scale_kernel(h_ref, w_ref, dinv_ref, out_ref):
    kk = pl.program_id(2)
    nk = pl.num_programs(2)

    @pl.when(kk == 0)
    def _():
        out_ref[...] = jnp.zeros_like(out_ref)

    out_ref[...] += jnp.dot(
        h_ref[...], w_ref[...], preferred_element_type=jnp.float32
    )[None]

    @pl.when(kk == nk - 1)
    def _():
        out_ref[...] *= dinv_ref[...][None]


def _mm_scale(h, W, dinv2d, kb):
    K = h.shape[1]
    grid = (N_NODES // MB, NHID // 128, K // kb)
    return pl.pallas_call(
        _mm_scale_kernel,
        grid=grid,
        in_specs=[
            pl.BlockSpec((MB, kb), lambda m, n, kk: (m, kk)),
            pl.BlockSpec((kb, 128), lambda m, n, kk: (kk, n)),
            pl.BlockSpec((MB, 1), lambda m, n, kk: (m, 0)),
        ],
        out_specs=pl.BlockSpec((1, MB, 128), lambda m, n, kk: (n, m, 0)),
        out_shape=jax.ShapeDtypeStruct((NCHUNK, N_NODES, 128), jnp.float32),
    )(h, W, dinv2d)


def _combine_kernel(agg_ref, zs_ref, dinv_ref, b_ref, out_ref):
    out_ref[...] = jax.nn.relu(
        dinv_ref[...] * (agg_ref[0] + zs_ref[0]) + b_ref[0]
    )


def _combine(agg, zs, dinv2d, b):
    return pl.pallas_call(
        _combine_kernel,
        grid=(N_NODES // MB, NHID // 128),
        in_specs=[
            pl.BlockSpec((1, MB, 128), lambda m, n: (n, m, 0)),
            pl.BlockSpec((1, MB, 128), lambda m, n: (n, m, 0)),
            pl.BlockSpec((MB, 1), lambda m, n: (m, 0)),
            pl.BlockSpec((1, 1, 128), lambda m, n: (n, 0, 0)),
        ],
        out_specs=pl.BlockSpec((MB, 128), lambda m, n: (m, n)),
        out_shape=jax.ShapeDtypeStruct((N_NODES, NHID), jnp.float32),
    )(agg, zs, dinv2d, b.reshape(NCHUNK, 1, 128))


def _pool_final_kernel(h_ref, batch_ref, wf_ref, bf_ref, out_ref, acc_ref, cnt_ref):
    m = pl.program_id(0)
    nm = pl.num_programs(0)

    @pl.when(m == 0)
    def _():
        acc_ref[...] = jnp.zeros_like(acc_ref)
        cnt_ref[...] = jnp.zeros_like(cnt_ref)

    h = h_ref[...]
    b = batch_ref[...]
    gids = jax.lax.broadcasted_iota(jnp.int32, (1, NUM_GRAPHS), 1)
    onehot = (b == gids).astype(jnp.float32)
    acc_ref[...] += jax.lax.dot_general(
        onehot, h, (((0,), (0,)), ((), ())), preferred_element_type=jnp.float32
    )
    cnt_ref[...] += jnp.sum(onehot, axis=0, keepdims=True)

    @pl.when(m == nm - 1)
    def _():
        cnt = jnp.maximum(cnt_ref[...], 1.0)
        g = acc_ref[...] / cnt.reshape(NUM_GRAPHS, 1)
        out_ref[...] = (
            jnp.dot(g, wf_ref[...], preferred_element_type=jnp.float32)
            + bf_ref[...]
        )


def _pool_final(h, batch, Wf, bf):
    return pl.pallas_call(
        _pool_final_kernel,
        grid=(N_NODES // MB,),
        in_specs=[
            pl.BlockSpec((MB, NHID), lambda m: (m, 0)),
            pl.BlockSpec((MB, 1), lambda m: (m, 0)),
            pl.BlockSpec((NHID, NCLASS), lambda m: (0, 0)),
            pl.BlockSpec((1, NCLASS), lambda m: (0, 0)),
        ],
        out_specs=pl.BlockSpec((NUM_GRAPHS, NCLASS), lambda m: (0, 0)),
        out_shape=jax.ShapeDtypeStruct((NUM_GRAPHS, NCLASS), jnp.float32),
        scratch_shapes=[
            pltpu.VMEM((NUM_GRAPHS, NHID), jnp.float32),
            pltpu.VMEM((1, NUM_GRAPHS), jnp.float32),
        ],
    )(h, batch.reshape(N_NODES, 1), Wf, bf.reshape(1, NCLASS))


def kernel(x, edge_index, edge_weight, batch, W1, b1, W2, b2, W3, b3, Wf, bf):
    src, dst = edge_index[0], edge_index[1]
    ew = edge_weight.astype(jnp.float32)

    npad = EPAD - N_EDGES
    fill = (jnp.arange(npad, dtype=jnp.int32) * 37) % N_NODES
    src2d = jnp.concatenate([src, fill]).reshape(ER, EW)
    dst2d = jnp.concatenate([dst, fill]).reshape(ER, EW)
    ew2d = jnp.concatenate([ew, jnp.zeros((npad,), jnp.float32)]).reshape(ER, EW)

    src_off = (src2d[None] +
               (jnp.arange(NCHUNK, dtype=jnp.int32) * N_NODES)[:, None, None])

    degp = _sc_deg(dst2d, ew2d)
    deg = 1.0 + degp[0, :N_NODES] + degp[1, :N_NODES]
    dinv2d = lax.rsqrt(deg).reshape(N_NODES, 1)

    h = x.astype(jnp.float32)
    for W, b, kb in ((W1, b1, 256), (W2, b2, 1024), (W3, b3, 1024)):
        zs = _mm_scale(h, W, dinv2d, kb)
        agg = _sc_agg(zs.reshape(NCHUNK * N_NODES, 128), src_off, dst2d,
                      ew2d)
        h = _combine(agg.reshape(NCHUNK, NP, 128), zs, dinv2d, b)

    return _pool_final(h, batch, Wf, bf)


# speedup vs baseline: 1.3610x; 1.0571x over previous
import functools

import jax
import jax.numpy as jnp
from jax import lax
from jax.experimental import pallas as pl
from jax.experimental.pallas import tpu as pltpu
from jax.experimental.pallas import tpu_sc as plsc

N_NODES = 10000
N_EDGES = 160000
NFEAT = 256
NHID = 1024
NCLASS = 64
NUM_GRAPHS = 64

NC, NS, LANES = 2, 16, 16

EPAD = 163840
EW = 64
ER = EPAD // EW
RPT = ER // NS
RPW = ER // (NC * NS)
BR = 32
DEPTH = 4
NCHUNK = NHID // 128
SLOTS = 4
CPSH = SLOTS // NC
NP = 10240
NROWS_T = NP // NS
ZR = 128
DEGP = 10240
DPT = DEGP // NS

MB = 2000

_sc_mesh = plsc.VectorSubcoreMesh(
    core_axis_name="c", subcore_axis_name="s", num_cores=NC, num_subcores=NS
)


def _sc_deg_body(dst_hbm, ew_hbm, out_hbm, accd, dstb, ewb, zbuf):
    c = lax.axis_index("c")
    s = lax.axis_index("s")
    wid = s * NC + c

    def _z(i, _):
        zbuf[pl.ds(i * LANES, LANES)] = jnp.zeros((LANES,), jnp.float32)
        return 0

    lax.fori_loop(0, DPT // LANES, _z, 0)
    pltpu.sync_copy(zbuf, accd.at[pl.ds(s * DPT, DPT)])
    plsc.subcore_barrier()

    pltpu.sync_copy(dst_hbm.at[pl.ds(wid * RPW, RPW)], dstb)
    pltpu.sync_copy(ew_hbm.at[pl.ds(wid * RPW, RPW)], ewb)

    def _row(r, _):
        pltpu.sync_copy(ewb.at[r], accd.at[dstb.at[r]], add=True)
        return 0

    lax.fori_loop(0, RPW, _row, 0)
    plsc.subcore_barrier()
    pltpu.sync_copy(accd.at[pl.ds(s * DPT, DPT)], out_hbm.at[c, pl.ds(s * DPT, DPT)])


@functools.partial(
    pl.kernel,
    out_type=jax.ShapeDtypeStruct((NC, DEGP), jnp.float32),
    mesh=_sc_mesh,
    scratch_types=[
        pltpu.VMEM_SHARED((DEGP,), jnp.float32),
        pltpu.VMEM((RPW, EW), jnp.int32),
        pltpu.VMEM((RPW, EW), jnp.float32),
        pltpu.VMEM((DPT,), jnp.float32),
    ],
)
def _sc_deg(dst_hbm, ew_hbm, out_hbm, accd, dstb, ewb, zbuf):
    _sc_deg_body(dst_hbm, ew_hbm, out_hbm, accd, dstb, ewb, zbuf)


_BCAST_DNUMS = lax.GatherDimensionNumbers(
    offset_dims=(), collapsed_slice_dims=(0,), start_index_map=(0,)
)


def _lane_bcast(v16, lane):
    idx = jnp.broadcast_to(lane, (LANES, 1)).astype(jnp.int32)
    return lax.gather(v16, idx, _BCAST_DNUMS, (1,),
                      mode=lax.GatherScatterMode.PROMISE_IN_BOUNDS)


def _sc_agg_body(zs_hbm, src_hbm, dst_hbm, ew_hbm, out_hbm,
                 acc, srcb, dstb, ewb, *bufs_and_sems):
    bufs = bufs_and_sems[:DEPTH]
    gsems = bufs_and_sems[DEPTH:2 * DEPTH]
    ssems = bufs_and_sems[2 * DEPTH:]
    c = lax.axis_index("c")
    s = lax.axis_index("s")

    def _scale(buf, ewblk, r):
        def _scale16(eb, _):
            w16 = ewblk[r, pl.ds(eb * LANES, LANES)]
            for l in range(LANES):
                wf = _lane_bcast(w16, l)
                e = eb * LANES + l
                for j in range(8):
                    sl = pl.ds(j * LANES, LANES)
                    buf[e, sl] = buf[e, sl] * wf
            return 0

        lax.fori_loop(0, EW // LANES, _scale16, 0)

    for ci in range(CPSH):
        chunk = c * CPSH + ci
        base = chunk * NP

        def _z(i, _):
            bufs[0][i // 8, pl.ds((i % 8) * LANES, LANES)] = jnp.zeros(
                (LANES,), jnp.float32)
            return 0

        lax.fori_loop(0, EW * 8, _z, 0)
        for p in range(NROWS_T // EW):
            pltpu.sync_copy(bufs[0], acc.at[pl.ds(s * NROWS_T + p * EW, EW)])
        plsc.subcore_barrier()

        def _blk(bi, _):
            gr0 = s * RPT + bi * BR
            pltpu.sync_copy(src_hbm.at[chunk, pl.ds(gr0, BR)], srcb)
            pltpu.sync_copy(dst_hbm.at[pl.ds(gr0, BR)], dstb)
            pltpu.sync_copy(ew_hbm.at[pl.ds(gr0, BR)], ewb)

            for b in range(DEPTH - 1):
                pltpu.async_copy(zs_hbm.at[srcb.at[b]], bufs[b], gsems[b])

            def _grp(q, _):
                for b in range(DEPTH):
                    r = DEPTH * q + b
                    nb = (b + DEPTH - 1) % DEPTH
                    pltpu.make_async_copy(
                        zs_hbm.at[srcb.at[r]], bufs[b], gsems[b]).wait()
                    _scale(bufs[b], ewb, r)
                    pltpu.async_copy(
                        bufs[b], acc.at[dstb.at[r]], ssems[b], add=True)

                    @pl.when(r + DEPTH - 1 < BR)
                    def _():
                        @pl.when(r > 0)
                        def _():
                            pltpu.make_async_copy(
                                bufs[nb], acc.at[dstb.at[r - 1]],
                                ssems[nb]).wait()

                        pltpu.async_copy(
                            zs_hbm.at[srcb.at[r + DEPTH - 1]], bufs[nb],
                            gsems[nb])

                return 0

            lax.fori_loop(0, BR // DEPTH, _grp, 0)
            for r in range(BR - DEPTH, BR):
                pltpu.make_async_copy(
                    bufs[r % DEPTH], acc.at[dstb.at[r]],
                    ssems[r % DEPTH]).wait()
            return 0

        lax.fori_loop(0, RPT // BR, _blk, 0)
        plsc.subcore_barrier()

        for p in range(5):
            row = s * NROWS_T + p * ZR
            pltpu.sync_copy(acc.at[pl.ds(row, ZR)],
                            out_hbm.at[pl.ds(base + row, ZR)])
        plsc.subcore_barrier()


@functools.partial(
    pl.kernel,
    out_type=jax.ShapeDtypeStruct((SLOTS * NP, 128), jnp.float32),
    mesh=_sc_mesh,
    scratch_types=(
        [
            pltpu.VMEM_SHARED((NP, 128), jnp.float32),
            pltpu.VMEM((BR, EW), jnp.int32),
            pltpu.VMEM((BR, EW), jnp.int32),
            pltpu.VMEM((BR, EW), jnp.float32),
        ]
        + [pltpu.VMEM((EW, 128), jnp.float32)] * DEPTH
        + [pltpu.SemaphoreType.DMA] * (2 * DEPTH)
    ),
)
def _sc_agg(zs_hbm, src_hbm, dst_hbm, ew_hbm, out_hbm,
            acc, srcb, dstb, ewb, *bufs_and_sems):
    _sc_agg_body(zs_hbm, src_hbm, dst_hbm, ew_hbm, out_hbm,
                 acc, srcb, dstb, ewb, *bufs_and_sems)


def _chk(s, half):
    return (s // 2) * 4 + (s % 2) + 2 * half


def _mm1_half_kernel(x_ref, w_ref, dinv_ref, out_ref):
    z = jnp.dot(x_ref[...], w_ref[...], preferred_element_type=jnp.float32)
    out_ref[...] = (dinv_ref[...] * z)[None]


def _make_mm1_half(half):
    return pl.pallas_call(
        _mm1_half_kernel,
        grid=(N_NODES // MB, SLOTS),
        in_specs=[
            pl.BlockSpec((MB, NFEAT), lambda m, n: (m, 0)),
            pl.BlockSpec((NFEAT, 128), lambda m, n: (0, _chk(n, half))),
            pl.BlockSpec((MB, 1), lambda m, n: (m, 0)),
        ],
        out_specs=pl.BlockSpec((1, MB, 128), lambda m, n: (n, m, 0)),
        out_shape=jax.ShapeDtypeStruct((SLOTS, N_NODES, 128), jnp.float32),
    )


_mm1_halves = (_make_mm1_half(0), _make_mm1_half(1))


def _make_mm23_half_kernel(half):
    def _k(ha_ref, hb_ref, w_ref, dinv_ref, out_ref):
        z = jnp.zeros((MB, 128), jnp.float32)
        for j in range(SLOTS):
            ra = _chk(j, 0) * 128
            rb = _chk(j, 1) * 128
            z += jnp.dot(ha_ref[j], w_ref[pl.ds(ra, 128), :],
                         preferred_element_type=jnp.float32)
            z += jnp.dot(hb_ref[j], w_ref[pl.ds(rb, 128), :],
                         preferred_element_type=jnp.float32)
        out_ref[...] = (dinv_ref[...] * z)[None]

    return _k


def _make_mm23_half(half):
    return pl.pallas_call(
        _make_mm23_half_kernel(half),
        grid=(N_NODES // MB, SLOTS),
        in_specs=[
            pl.BlockSpec((SLOTS, MB, 128), lambda m, n: (0, m, 0)),
            pl.BlockSpec((SLOTS, MB, 128), lambda m, n: (0, m, 0)),
            pl.BlockSpec((NHID, 128), lambda m, n: (0, _chk(n, half))),
            pl.BlockSpec((MB, 1), lambda m, n: (m, 0)),
        ],
        out_specs=pl.BlockSpec((1, MB, 128), lambda m, n: (n, m, 0)),
        out_shape=jax.ShapeDtypeStruct((SLOTS, N_NODES, 128), jnp.float32),
    )


_mm23_halves = (_make_mm23_half(0), _make_mm23_half(1))


def _combine_kernel(agg_ref, zs_ref, dinv_ref, b_ref, out_ref):
    out_ref[...] = jax.nn.relu(
        dinv_ref[...] * (agg_ref[0] + zs_ref[0]) + b_ref[0]
    )[None]


def _make_combine_half(half):
    return pl.pallas_call(
        _combine_kernel,
        grid=(N_NODES // MB, SLOTS),
        in_specs=[
            pl.BlockSpec((1, MB, 128), lambda m, n: (n, m, 0)),
            pl.BlockSpec((1, MB, 128), lambda m, n: (n, m, 0)),
            pl.BlockSpec((MB, 1), lambda m, n: (m, 0)),
            pl.BlockSpec((1, 1, 128), lambda m, n: (_chk(n, half), 0, 0)),
        ],
        out_specs=pl.BlockSpec((1, MB, 128), lambda m, n: (n, m, 0)),
        out_shape=jax.ShapeDtypeStruct((SLOTS, N_NODES, 128), jnp.float32),
    )


_combine_halves = (_make_combine_half(0), _make_combine_half(1))


def _pool_final_kernel(ha_ref, hb_ref, batch_ref, wf_ref, bf_ref, out_ref,
                       acc_ref, cnt_ref):
    m = pl.program_id(0)
    nm = pl.num_programs(0)

    @pl.when(m == 0)
    def _():
        acc_ref[...] = jnp.zeros_like(acc_ref)
        cnt_ref[...] = jnp.zeros_like(cnt_ref)

    b = batch_ref[...]
    gids = jax.lax.broadcasted_iota(jnp.int32, (1, NUM_GRAPHS), 1)
    onehot = (b == gids).astype(jnp.float32)
    for j in range(SLOTS):
        acc_ref[_chk(j, 0)] += jax.lax.dot_general(
            onehot, ha_ref[j], (((0,), (0,)), ((), ())),
            preferred_element_type=jnp.float32)
        acc_ref[_chk(j, 1)] += jax.lax.dot_general(
            onehot, hb_ref[j], (((0,), (0,)), ((), ())),
            preferred_element_type=jnp.float32)
    cnt_ref[...] += jnp.sum(onehot, axis=0, keepdims=True)

    @pl.when(m == nm - 1)
    def _():
        inv = 1.0 / jnp.maximum(cnt_ref[...], 1.0)
        out = bf_ref[...]
        for q in range(NCHUNK):
            g = acc_ref[q] * inv.reshape(NUM_GRAPHS, 1)
            out += jnp.dot(g, wf_ref[pl.ds(q * 128, 128), :],
                           preferred_element_type=jnp.float32)
        out_ref[...] = out


def _pool_final(ha, hb, batch, Wf, bf):
    return pl.pallas_call(
        _pool_final_kernel,
        grid=(N_NODES // MB,),
        in_specs=[
            pl.BlockSpec((SLOTS, MB, 128), lambda m: (0, m, 0)),
            pl.BlockSpec((SLOTS, MB, 128), lambda m: (0, m, 0)),
            pl.BlockSpec((MB, 1), lambda m: (m, 0)),
            pl.BlockSpec((NHID, NCLASS), lambda m: (0, 0)),
            pl.BlockSpec((1, NCLASS), lambda m: (0, 0)),
        ],
        out_specs=pl.BlockSpec((NUM_GRAPHS, NCLASS), lambda m: (0, 0)),
        out_shape=jax.ShapeDtypeStruct((NUM_GRAPHS, NCLASS), jnp.float32),
        scratch_shapes=[
            pltpu.VMEM((NCHUNK, NUM_GRAPHS, 128), jnp.float32),
            pltpu.VMEM((1, NUM_GRAPHS), jnp.float32),
        ],
    )(ha, hb, batch.reshape(N_NODES, 1), Wf, bf.reshape(1, NCLASS))


def kernel(x, edge_index, edge_weight, batch, W1, b1, W2, b2, W3, b3, Wf, bf):
    src, dst = edge_index[0], edge_index[1]
    ew = edge_weight.astype(jnp.float32)

    npad = EPAD - N_EDGES
    fill = (jnp.arange(npad, dtype=jnp.int32) * 37) % N_NODES
    src2d = jnp.concatenate([src, fill]).reshape(ER, EW)
    dst2d = jnp.concatenate([dst, fill]).reshape(ER, EW)
    ew2d = jnp.concatenate([ew, jnp.zeros((npad,), jnp.float32)]).reshape(ER, EW)

    src_off = (src2d[None] +
               (jnp.arange(SLOTS, dtype=jnp.int32) * N_NODES)[:, None, None])

    degp = _sc_deg(dst2d, ew2d)
    deg = 1.0 + degp[0, :N_NODES] + degp[1, :N_NODES]
    dinv2d = lax.rsqrt(deg).reshape(N_NODES, 1)

    x = x.astype(jnp.float32)
    ha = hb = None
    for li, (W, b) in enumerate(((W1, b1), (W2, b2), (W3, b3))):
        if li == 0:
            zsa = _mm1_halves[0](x, W, dinv2d)
            zsb = _mm1_halves[1](x, W, dinv2d)
        else:
            zsa = _mm23_halves[0](ha, hb, W, dinv2d)
            zsb = _mm23_halves[1](ha, hb, W, dinv2d)
        agga = _sc_agg(zsa.reshape(SLOTS * N_NODES, 128), src_off, dst2d, ew2d)
        aggb = _sc_agg(zsb.reshape(SLOTS * N_NODES, 128), src_off, dst2d, ew2d)
        ha = _combine_halves[0](agga.reshape(SLOTS, NP, 128), zsa, dinv2d,
                                b.reshape(NCHUNK, 1, 128))
        hb = _combine_halves[1](aggb.reshape(SLOTS, NP, 128), zsb, dinv2d,
                                b.reshape(NCHUNK, 1, 128))

    return _pool_final(ha, hb, batch, Wf, bf)
